# Initial kernel scaffold; baseline (speedup 1.0000x reference)
import jax
import jax.numpy as jnp
from jax.experimental import pallas as pl


def kernel(x, membership, edges, weights, W_rel1, b_rel1, W_root1, W_rel2, b_rel2, W_root2, bn1_g, bn1_b, bn2_g, bn2_b, fc1_W, fc1_b, bn3_g, bn3_b, fc2_W, fc2_b):
    raise NotImplementedError("write your pallas kernel here")


# speedup vs baseline: 3.8863x; 3.8863x over previous
import functools

import jax
import jax.numpy as jnp
from jax import lax
from jax.experimental import pallas as pl
from jax.experimental.pallas import tpu as pltpu
from jax.experimental.pallas import tpu_sc as plsc

N = 10000
E = 320000
D = 128
G = 64
FC = 256
OUT = 10

NC = 2
NS = 16
NW = NC * NS
EPT = E // NW
CHUNK = 80
NCHUNK = EPT // CHUNK
RPT = 624
RPT0 = 16
NLANE = D // 16


def _make_spmm(interpret=False):
  mesh = plsc.VectorSubcoreMesh(core_axis_name="c", subcore_axis_name="s")

  @functools.partial(
      pl.kernel,
      out_type=jax.ShapeDtypeStruct((NC, N, D), jnp.float32),
      mesh=mesh,
      scratch_types=[
          pltpu.VMEM((CHUNK,), jnp.int32),
          pltpu.VMEM((CHUNK,), jnp.int32),
          pltpu.VMEM((CHUNK,), jnp.float32),
          pltpu.VMEM((CHUNK, D), jnp.float32),
          pltpu.VMEM_SHARED((N, D), jnp.float32),
          pltpu.SemaphoreType.DMA,
      ],
      interpret=interpret,
  )
  def spmm(x_hbm, src_hbm, dst_hbm, w_hbm, out_hbm,
           src_v, dst_v, w_v, rows_v, acc, sem):
    c = lax.axis_index("c")
    s = lax.axis_index("s")
    wid = c * NS + s

    zero16 = jnp.zeros((16,), jnp.float32)

    @pl.loop(0, CHUNK)
    def _zero_fill(r):
      for j in range(NLANE):
        rows_v[r, pl.ds(j * 16, 16)] = zero16

    row0 = RPT0 + s * RPT
    for k in range(RPT // CHUNK):
      pltpu.sync_copy(rows_v, acc.at[pl.ds(row0 + k * CHUNK, CHUNK)])
    rem = RPT - (RPT // CHUNK) * CHUNK
    pltpu.sync_copy(rows_v.at[pl.ds(0, rem)],
                    acc.at[pl.ds(row0 + RPT - rem, rem)])

    @pl.when(s == 0)
    def _zero_head():
      pltpu.sync_copy(rows_v.at[pl.ds(0, RPT0)], acc.at[pl.ds(0, RPT0)])

    plsc.subcore_barrier()

    @pl.loop(0, NCHUNK)
    def _edge_chunk(ci):
      base = wid * EPT + ci * CHUNK
      pltpu.sync_copy(src_hbm.at[pl.ds(base, CHUNK)], src_v)
      pltpu.sync_copy(dst_hbm.at[pl.ds(base, CHUNK)], dst_v)
      pltpu.sync_copy(w_hbm.at[pl.ds(base, CHUNK)], w_v)
      pltpu.async_copy(x_hbm.at[src_v], rows_v, sem).wait()

      @pl.loop(0, CHUNK // 16)
      def _scale(g):
        wgrp = w_v[pl.ds(g * 16, 16)]
        for e in range(16):
          wv = wgrp.at[jnp.full((16,), e, jnp.int32)].get(
              mode="promise_in_bounds")
          i = g * 16 + e
          for j in range(NLANE):
            sl = pl.ds(j * 16, 16)
            rows_v[i, sl] = rows_v[i, sl] * wv

      pltpu.sync_copy(rows_v, acc.at[dst_v], add=True)

    plsc.subcore_barrier()
    pltpu.sync_copy(acc.at[pl.ds(row0, RPT)],
                    out_hbm.at[c, pl.ds(row0, RPT)])

    @pl.when(s == 0)
    def _write_head():
      pltpu.sync_copy(acc.at[pl.ds(0, RPT0)], out_hbm.at[c, pl.ds(0, RPT0)])

  return spmm


def _dense1_body(aggp, x, wrel, brel, wroot, g, b, out):
  agg = aggp[0] + aggp[1]
  z = lax.dot_general(agg, wrel[...], (((1,), (1,)), ((), ())),
                      precision=lax.Precision.HIGHEST)
  z = z + brel[...]
  z = z + lax.dot_general(x[...], wroot[...], (((1,), (1,)), ((), ())),
                          precision=lax.Precision.HIGHEST)
  z = jnp.maximum(z, 0.0)
  mu = jnp.mean(z, axis=0, keepdims=True)
  var = jnp.mean((z - mu) ** 2, axis=0, keepdims=True)
  out[...] = g[...] * (z - mu) / jnp.sqrt(var + 1e-5) + b[...]


def _dense2_body(aggp, h1, wrel, brel, wroot, g2, b2, mem2d,
                 fc1w, fc1b, g3, b3, fc2w, fc2b, out):
  agg = aggp[0] + aggp[1]
  z = lax.dot_general(agg, wrel[...], (((1,), (1,)), ((), ())),
                      precision=lax.Precision.HIGHEST)
  z = z + brel[...]
  z = z + lax.dot_general(h1[...], wroot[...], (((1,), (1,)), ((), ())),
                          precision=lax.Precision.HIGHEST)
  z = jnp.maximum(z, 0.0)
  mu = jnp.mean(z, axis=0, keepdims=True)
  var = jnp.mean((z - mu) ** 2, axis=0, keepdims=True)
  h2 = g2[...] * (z - mu) / jnp.sqrt(var + 1e-5) + b2[...]

  gids = lax.broadcasted_iota(jnp.int32, (N, G), 1)
  oh = (mem2d[...] == gids).astype(jnp.float32)
  ssum = lax.dot_general(oh, h2, (((0,), (0,)), ((), ())),
                         precision=lax.Precision.HIGHEST)
  cnt = lax.dot_general(oh, jnp.ones((N, 1), jnp.float32),
                        (((0,), (0,)), ((), ())),
                        precision=lax.Precision.HIGHEST)
  pooled = ssum / jnp.maximum(cnt, 1.0)

  a = lax.dot_general(pooled, fc1w[...], (((1,), (1,)), ((), ())),
                      precision=lax.Precision.HIGHEST)
  a = jnp.maximum(a + fc1b[...], 0.0)
  mu3 = jnp.mean(a, axis=0, keepdims=True)
  var3 = jnp.mean((a - mu3) ** 2, axis=0, keepdims=True)
  a = g3[...] * (a - mu3) / jnp.sqrt(var3 + 1e-5) + b3[...]

  logits = lax.dot_general(a, fc2w[...], (((1,), (1,)), ((), ())),
                           precision=lax.Precision.HIGHEST)
  out[...] = logits + fc2b[...]


_TC_PARAMS = pltpu.CompilerParams(vmem_limit_bytes=100 * 1024 * 1024)


def _make_dense1(interpret=False):
  return pl.pallas_call(
      _dense1_body,
      out_shape=jax.ShapeDtypeStruct((N, D), jnp.float32),
      compiler_params=_TC_PARAMS,
      interpret=interpret,
  )


def _make_dense2(interpret=False):
  return pl.pallas_call(
      _dense2_body,
      out_shape=jax.ShapeDtypeStruct((G, OUT), jnp.float32),
      compiler_params=_TC_PARAMS,
      interpret=interpret,
  )


def kernel(x, membership, edges, weights, W_rel1, b_rel1, W_root1,
           W_rel2, b_rel2, W_root2, bn1_g, bn1_b, bn2_g, bn2_b,
           fc1_W, fc1_b, bn3_g, bn3_b, fc2_W, fc2_b):
  src = edges[0]
  dst = edges[1]
  mem2d = membership.reshape(N, 1)

  spmm = _make_spmm()
  dense1 = _make_dense1()
  dense2 = _make_dense2()

  aggp1 = spmm(x, src, dst, weights)
  h1 = dense1(aggp1, x, W_rel1, b_rel1.reshape(1, D), W_root1,
              bn1_g.reshape(1, D), bn1_b.reshape(1, D))
  aggp2 = spmm(h1, src, dst, weights)
  logits = dense2(aggp2, h1, W_rel2, b_rel2.reshape(1, D), W_root2,
                  bn2_g.reshape(1, D), bn2_b.reshape(1, D), mem2d,
                  fc1_W, fc1_b.reshape(1, FC), bn3_g.reshape(1, FC),
                  bn3_b.reshape(1, FC), fc2_W, fc2_b.reshape(1, OUT))
  return logits


# speedup vs baseline: 9.7138x; 2.4995x over previous
import functools

import jax
import jax.numpy as jnp
from jax import lax
from jax.experimental import pallas as pl
from jax.experimental.pallas import tpu as pltpu
from jax.experimental.pallas import tpu_sc as plsc

N = 10000
E = 320000
D = 128
G = 64
FC = 256
OUT = 10

NC = 2
NS = 16
NW = NC * NS
EPT = E // NW
CHUNK = 80
NCHUNK = EPT // CHUNK
RPT = 624
RPT0 = 16
NLANE = D // 16


def _make_spmm(interpret=False):
  mesh = plsc.VectorSubcoreMesh(core_axis_name="c", subcore_axis_name="s")

  @functools.partial(
      pl.kernel,
      out_type=jax.ShapeDtypeStruct((NC, N, D), jnp.float32),
      mesh=mesh,
      scratch_types=[
          pltpu.VMEM((EPT,), jnp.int32),
          pltpu.VMEM((EPT,), jnp.float32),
          pltpu.VMEM((CHUNK,), jnp.int32),
          pltpu.VMEM((CHUNK,), jnp.int32),
          pltpu.VMEM((CHUNK,), jnp.int32),
          pltpu.VMEM((CHUNK,), jnp.int32),
          pltpu.VMEM((CHUNK, D), jnp.float32),
          pltpu.VMEM((CHUNK, D), jnp.float32),
          pltpu.VMEM_SHARED((N, D), jnp.float32),
          pltpu.SemaphoreType.DMA,
          pltpu.SemaphoreType.DMA,
          pltpu.SemaphoreType.DMA,
      ],
      interpret=interpret,
  )
  def spmm(x_hbm, pk_hbm, w_hbm, out_hbm,
           pk_v, w_v, srcA, srcB, dstA, dstB, rowsA, rowsB, acc,
           semi, semgA, semgB):
    c = lax.axis_index("c")
    s = lax.axis_index("s")
    wid = c * NS + s
    ebase = wid * EPT

    dpk = pltpu.async_copy(pk_hbm.at[pl.ds(ebase, EPT)], pk_v, semi)
    dw = pltpu.async_copy(w_hbm.at[pl.ds(ebase, EPT)], w_v, semi)

    zero16 = jnp.zeros((16,), jnp.float32)
    srcc = (srcA, srcB)
    dstc = (dstA, dstB)
    rows = (rowsA, rowsB)
    semg = (semgA, semgB)

    @pl.loop(0, CHUNK)
    def _zero_fill(r):
      for j in range(NLANE):
        rowsA[r, pl.ds(j * 16, 16)] = zero16

    row0 = RPT0 + s * RPT
    zcopies = []
    for k in range(RPT // CHUNK):
      zcopies.append(pltpu.async_copy(
          rowsA, acc.at[pl.ds(row0 + k * CHUNK, CHUNK)], semgA))
    rem = RPT - (RPT // CHUNK) * CHUNK
    zcopies.append(pltpu.async_copy(
        rowsA.at[pl.ds(0, rem)], acc.at[pl.ds(row0 + RPT - rem, rem)],
        semgA))

    @pl.when(s == 0)
    def _zero_head():
      pltpu.sync_copy(rowsA.at[pl.ds(0, RPT0)], acc.at[pl.ds(0, RPT0)])

    for d in zcopies:
      d.wait()
    dpk.wait()
    dw.wait()
    plsc.subcore_barrier()

    def unpack(k, r):
      for g in range(CHUNK // 16):
        v = pk_v[pl.ds(k * CHUNK + g * 16, 16)]
        sl = pl.ds(g * 16, 16)
        srcc[r][sl] = jnp.bitwise_and(v, 0xFFFF)
        dstc[r][sl] = lax.shift_right_logical(v, 16)

    def issue_gather(r):
      return pltpu.async_copy(x_hbm.at[srcc[r]], rows[r], semg[r])

    def wait_gather(r):
      pltpu.make_async_copy(x_hbm.at[pl.ds(0, CHUNK)], rows[r],
                            semg[r]).wait()

    def scale_scatter(k, r):
      rbuf = rows[r]

      @pl.loop(0, CHUNK // 16)
      def _scale(g):
        wgrp = w_v[pl.ds(k * CHUNK + g * 16, 16)]
        for e in range(16):
          wv = wgrp.at[jnp.full((16,), e, jnp.int32)].get(
              mode="promise_in_bounds")
          i = g * 16 + e
          for j in range(NLANE):
            sl = pl.ds(j * 16, 16)
            rbuf[i, sl] = rbuf[i, sl] * wv

      pltpu.sync_copy(rbuf, acc.at[dstc[r]], add=True)

    unpack(0, 0)
    issue_gather(0)

    @pl.loop(0, (NCHUNK - 1) // 2)
    def _pair(i):
      k0 = i * 2
      unpack(k0 + 1, 1)
      issue_gather(1)
      wait_gather(0)
      scale_scatter(k0, 0)
      unpack(k0 + 2, 0)
      issue_gather(0)
      wait_gather(1)
      scale_scatter(k0 + 1, 1)

    wait_gather(0)
    scale_scatter(NCHUNK - 1, 0)

    plsc.subcore_barrier()
    pltpu.sync_copy(acc.at[pl.ds(row0, RPT)],
                    out_hbm.at[c, pl.ds(row0, RPT)])

    @pl.when(s == 0)
    def _write_head():
      pltpu.sync_copy(acc.at[pl.ds(0, RPT0)], out_hbm.at[c, pl.ds(0, RPT0)])

  return spmm


def _dense1_body(aggp, x, wrel, brel, wroot, g, b, out):
  agg = aggp[0] + aggp[1]
  z = lax.dot_general(agg, wrel[...], (((1,), (1,)), ((), ())),
                      precision=lax.Precision.HIGHEST)
  z = z + brel[...]
  z = z + lax.dot_general(x[...], wroot[...], (((1,), (1,)), ((), ())),
                          precision=lax.Precision.HIGHEST)
  z = jnp.maximum(z, 0.0)
  mu = jnp.mean(z, axis=0, keepdims=True)
  var = jnp.mean((z - mu) ** 2, axis=0, keepdims=True)
  out[...] = g[...] * (z - mu) / jnp.sqrt(var + 1e-5) + b[...]


def _dense2_body(aggp, h1, wrel, brel, wroot, g2, b2, mem2d,
                 fc1w, fc1b, g3, b3, fc2w, fc2b, out):
  agg = aggp[0] + aggp[1]
  z = lax.dot_general(agg, wrel[...], (((1,), (1,)), ((), ())),
                      precision=lax.Precision.HIGHEST)
  z = z + brel[...]
  z = z + lax.dot_general(h1[...], wroot[...], (((1,), (1,)), ((), ())),
                          precision=lax.Precision.HIGHEST)
  z = jnp.maximum(z, 0.0)
  mu = jnp.mean(z, axis=0, keepdims=True)
  var = jnp.mean((z - mu) ** 2, axis=0, keepdims=True)
  h2 = g2[...] * (z - mu) / jnp.sqrt(var + 1e-5) + b2[...]

  gids = lax.broadcasted_iota(jnp.int32, (N, G), 1)
  oh = (mem2d[...] == gids).astype(jnp.float32)
  ssum = lax.dot_general(oh, h2, (((0,), (0,)), ((), ())),
                         precision=lax.Precision.HIGHEST)
  cnt = lax.dot_general(oh, jnp.ones((N, 1), jnp.float32),
                        (((0,), (0,)), ((), ())),
                        precision=lax.Precision.HIGHEST)
  pooled = ssum / jnp.maximum(cnt, 1.0)

  a = lax.dot_general(pooled, fc1w[...], (((1,), (1,)), ((), ())),
                      precision=lax.Precision.HIGHEST)
  a = jnp.maximum(a + fc1b[...], 0.0)
  mu3 = jnp.mean(a, axis=0, keepdims=True)
  var3 = jnp.mean((a - mu3) ** 2, axis=0, keepdims=True)
  a = g3[...] * (a - mu3) / jnp.sqrt(var3 + 1e-5) + b3[...]

  logits = lax.dot_general(a, fc2w[...], (((1,), (1,)), ((), ())),
                           precision=lax.Precision.HIGHEST)
  out[...] = logits + fc2b[...]


_TC_PARAMS = pltpu.CompilerParams(vmem_limit_bytes=100 * 1024 * 1024)


def _make_dense1(interpret=False):
  return pl.pallas_call(
      _dense1_body,
      out_shape=jax.ShapeDtypeStruct((N, D), jnp.float32),
      compiler_params=_TC_PARAMS,
      interpret=interpret,
  )


def _make_dense2(interpret=False):
  return pl.pallas_call(
      _dense2_body,
      out_shape=jax.ShapeDtypeStruct((G, OUT), jnp.float32),
      compiler_params=_TC_PARAMS,
      interpret=interpret,
  )


def kernel(x, membership, edges, weights, W_rel1, b_rel1, W_root1,
           W_rel2, b_rel2, W_root2, bn1_g, bn1_b, bn2_g, bn2_b,
           fc1_W, fc1_b, bn3_g, bn3_b, fc2_W, fc2_b):
  packed = jnp.bitwise_or(edges[0], jnp.left_shift(edges[1], 16))
  mem2d = membership.reshape(N, 1)

  spmm = _make_spmm()
  dense1 = _make_dense1()
  dense2 = _make_dense2()

  aggp1 = spmm(x, packed, weights)
  h1 = dense1(aggp1, x, W_rel1, b_rel1.reshape(1, D), W_root1,
              bn1_g.reshape(1, D), bn1_b.reshape(1, D))
  aggp2 = spmm(h1, packed, weights)
  logits = dense2(aggp2, h1, W_rel2, b_rel2.reshape(1, D), W_root2,
                  bn2_g.reshape(1, D), bn2_b.reshape(1, D), mem2d,
                  fc1_W, fc1_b.reshape(1, FC), bn3_g.reshape(1, FC),
                  bn3_b.reshape(1, FC), fc2_W, fc2_b.reshape(1, OUT))
  return logits
